# baseline (speedup 1.0000x reference)
import jax
import jax.numpy as jnp
from jax import lax
from jax.experimental import pallas as pl
from jax.experimental.pallas import tpu as pltpu

_B, _S, _T, _M = 16, 512, 2048, 80
_LW = 640
_GR = _LW // _M
_ROWS = _B * _T * _M // _LW
_BR = 512
_NSTEP = _ROWS // _BR


def _body(melt_hbm, melp_hbm, pn_hbm, v_ref, oh_ref, pp_ref, pt_ref, ep_ref,
          et_ref, lp_ref, dt_ref, sm_ref, out_ref,
          bt_ref, bp_ref, bn_ref, st_ref, sp_ref, sn_ref):

    def _copies(step, slot):
        sl = pl.ds(step * _BR, _BR)
        return (
            pltpu.make_async_copy(melt_hbm.at[sl], bt_ref.at[slot], st_ref.at[slot]),
            pltpu.make_async_copy(melp_hbm.at[sl], bp_ref.at[slot], sp_ref.at[slot]),
            pltpu.make_async_copy(pn_hbm.at[sl], bn_ref.at[slot], sn_ref.at[slot]),
        )

    for c in _copies(0, 0):
        c.start()

    acc_mel = jnp.float32(0.0)
    acc_pn = jnp.float32(0.0)
    for i in range(_NSTEP):
        slot = i % 2
        if i + 1 < _NSTEP:
            for c in _copies(i + 1, 1 - slot):
                c.start()
        for c in _copies(i, slot):
            c.wait()
        mexp = lax.dot_general(
            v_ref[i * _BR:(i + 1) * _BR, :], oh_ref[...],
            (((1,), (0,)), ((), ())),
            precision=lax.Precision.HIGHEST,
            preferred_element_type=jnp.float32)
        t = bt_ref[slot]
        acc_mel += jnp.sum(jnp.abs(bp_ref[slot] - t) * mexp)
        acc_pn += jnp.sum(jnp.abs(bn_ref[slot] - t) * mexp)

    srcv = sm_ref[...]
    dp = pp_ref[...] - pt_ref[...]
    de = ep_ref[...] - et_ref[...]
    dd = lp_ref[...] - jnp.log(dt_ref[...] + 1.0)
    out_ref[0] = acc_mel
    out_ref[1] = acc_pn
    out_ref[2] = jnp.sum(v_ref[...])
    out_ref[3] = jnp.sum(dp * dp * srcv)
    out_ref[4] = jnp.sum(de * de * srcv)
    out_ref[5] = jnp.sum(dd * dd * srcv)
    out_ref[6] = jnp.sum(srcv)


def _losses(mel_t, mel_p, pn_p, valid_f, onehot, pitch_p, pitch_t, energy_p,
            energy_t, logdur_p, dur_f, src_valid):
    hbm = pl.BlockSpec(memory_space=pltpu.MemorySpace.HBM)
    vm = pl.BlockSpec(memory_space=pltpu.VMEM)
    buf = pltpu.VMEM((2, _BR, _LW), jnp.float32)
    sem = pltpu.SemaphoreType.DMA((2,))
    return pl.pallas_call(
        _body,
        in_specs=[hbm, hbm, hbm, vm, vm, vm, vm, vm, vm, vm, vm, vm],
        out_specs=pl.BlockSpec(memory_space=pltpu.SMEM),
        out_shape=jax.ShapeDtypeStruct((7,), jnp.float32),
        scratch_shapes=[buf, buf, buf, sem, sem, sem],
    )(mel_t, mel_p, pn_p, valid_f, onehot, pitch_p, pitch_t, energy_p,
      energy_t, logdur_p, dur_f, src_valid)


def kernel(mel_targets, pitch_targets, energy_targets, duration_targets,
           mel_predictions, postnet_mel_predictions, pitch_predictions,
           energy_predictions, log_duration_predictions, src_masks,
           mel_masks):
    valid_f = (~mel_masks).astype(jnp.float32).reshape(_ROWS, _GR)
    src_valid = (~src_masks).astype(jnp.float32)
    dur_f = duration_targets.astype(jnp.float32)
    onehot = (jnp.arange(_LW, dtype=jnp.int32)[None, :] // _M
              == jnp.arange(_GR, dtype=jnp.int32)[:, None]
              ).astype(jnp.float32)

    sums = _losses(mel_targets.reshape(_ROWS, _LW),
                   mel_predictions.reshape(_ROWS, _LW),
                   postnet_mel_predictions.reshape(_ROWS, _LW),
                   valid_f, onehot, pitch_predictions, pitch_targets,
                   energy_predictions, energy_targets,
                   log_duration_predictions, dur_f, src_valid)

    mel_den = jnp.maximum(sums[2] * _M, 1.0)
    src_den = jnp.maximum(sums[6], 1.0)
    mel_loss = sums[0] / mel_den
    postnet_mel_loss = sums[1] / mel_den
    pitch_loss = sums[3] / src_den
    energy_loss = sums[4] / src_den
    duration_loss = sums[5] / src_den
    total_loss = (mel_loss + postnet_mel_loss + duration_loss + pitch_loss
                  + energy_loss)
    return (total_loss, mel_loss, postnet_mel_loss, pitch_loss, energy_loss,
            duration_loss)
